# baseline (speedup 1.0000x reference)
import functools

import jax
import jax.numpy as jnp
from jax import lax
from jax.experimental import pallas as pl
from jax.experimental.pallas import tpu as pltpu
from jax.experimental.pallas import tpu_sc as plsc

_N = 10000
_E = 160000
_D = 256
_HALF = _D // 2
_CHUNK = 64
_NSUB = 16
_NCORE = 2
_CPT = 160
_ECHUNKS = _CPT * _NSUB
_EPAD = _ECHUNKS * _CHUNK
_NPAD = 10240
_TRASH = _NPAD - 8
_STRIPE = _NPAD // _NSUB


def _sc_body(x2, edges, zagg, zcnt, ones_h,
             agg_out, cnt_out,
             src_v, dst_v, rows_v, ones_v, agg_sh, cnt_sh,
             sem_g0, sem_g1, sem_g2, sem_g3,
             sem_d0, sem_d1, sem_d2, sem_d3):
    sem_g = (sem_g0, sem_g1, sem_g2, sem_g3)
    sem_d = (sem_d0, sem_d1, sem_d2, sem_d3)
    c = lax.axis_index("c")
    s = lax.axis_index("s")
    row0 = s * _STRIPE
    stripe = pl.ds(row0, _STRIPE)
    e0 = s * _CPT * _CHUNK

    pltpu.sync_copy(zagg.at[stripe], agg_sh.at[stripe])
    pltpu.sync_copy(zcnt.at[stripe], cnt_sh.at[stripe])
    pltpu.sync_copy(ones_h, ones_v)
    pltpu.sync_copy(edges.at[pl.ds(e0, _CPT * _CHUNK)], src_v)

    plsc.subcore_barrier()

    def gather(k, slot):
        return pltpu.make_async_copy(
            x2.at[src_v.at[pl.ds(k * _CHUNK, _CHUNK)]], rows_v.at[slot],
            sem_g[slot])

    def dst_load(k, slot):
        return pltpu.make_async_copy(
            edges.at[pl.ds(_EPAD + e0 + k * _CHUNK, _CHUNK)],
            dst_v.at[slot], sem_d[slot])

    def start(k, slot):
        dst_load(k, slot).start()
        gather(k, slot).start()

    def service(k, slot):
        gather(k, slot).wait()
        dst_load(k, slot).wait()

    start(0, 0)
    start(1, 1)

    def group(g, carry):
        for u in range(2):
            k = 2 * g + u
            service(k, u)
            if True:
                pass
            start(k + 2, u)
        return carry

    lax.fori_loop(0, (_CPT - 2) // 2, group, 0)
    service(_CPT - 2, 0)
    service(_CPT - 1, 1)

    plsc.subcore_barrier()

    pltpu.sync_copy(agg_sh.at[stripe],
                    agg_out.at[pl.ds(c * _NPAD + row0, _STRIPE)])
    pltpu.sync_copy(cnt_sh.at[stripe],
                    cnt_out.at[pl.ds(c * _NPAD + row0, _STRIPE)])


_sc_call = functools.partial(
    pl.kernel,
    out_type=(
        jax.ShapeDtypeStruct((_NCORE * _NPAD, _HALF), jnp.float32),
        jax.ShapeDtypeStruct((_NCORE * _NPAD,), jnp.float32),
    ),
    mesh=plsc.VectorSubcoreMesh(core_axis_name="c", subcore_axis_name="s"),
    scratch_types=[
        pltpu.VMEM((_CPT * _CHUNK,), jnp.int32),
        pltpu.VMEM((4, _CHUNK), jnp.int32),
        pltpu.VMEM((2, _CHUNK, _D), jnp.float32),
        pltpu.VMEM((_CHUNK,), jnp.float32),
        pltpu.VMEM_SHARED((_NPAD, _HALF), jnp.float32),
        pltpu.VMEM_SHARED((_NPAD,), jnp.float32),
        pltpu.SemaphoreType.DMA,
        pltpu.SemaphoreType.DMA,
        pltpu.SemaphoreType.DMA,
        pltpu.SemaphoreType.DMA,
        pltpu.SemaphoreType.DMA,
        pltpu.SemaphoreType.DMA,
        pltpu.SemaphoreType.DMA,
        pltpu.SemaphoreType.DMA,
    ],
)(_sc_body)


_BLK = 1000


def _tc_body(x_ref, a0_ref, a1_ref, cnt_ref, wl_ref, bl_ref, wr_ref,
             wlin_ref, blin_ref, o_ref):
    inv = 1.0 / jnp.maximum(cnt_ref[...], 1.0)
    m0 = a0_ref[...] * inv
    m1 = a1_ref[...] * inv
    h = (jnp.dot(m0, wl_ref[0:_HALF, :], preferred_element_type=jnp.float32)
         + jnp.dot(m1, wl_ref[_HALF:, :], preferred_element_type=jnp.float32)
         + jnp.dot(x_ref[...], wr_ref[...], preferred_element_type=jnp.float32)
         + bl_ref[...])
    h = jnp.maximum(h, 0.0)
    o_ref[...] = jnp.dot(h, wlin_ref[...],
                         preferred_element_type=jnp.float32) + blin_ref[...]


def _tc_tail(x, a0, a1, cnt, W_l, b_l, W_r, W_lin, b_lin):
    grid = (_N // _BLK,)
    return pl.pallas_call(
        _tc_body,
        grid=grid,
        in_specs=[
            pl.BlockSpec((_BLK, _D), lambda i: (i, 0)),
            pl.BlockSpec((_BLK, _HALF), lambda i: (i, 0)),
            pl.BlockSpec((_BLK, _HALF), lambda i: (i, 0)),
            pl.BlockSpec((_BLK, 1), lambda i: (i, 0)),
            pl.BlockSpec((_D, _D), lambda i: (0, 0)),
            pl.BlockSpec((1, _D), lambda i: (0, 0)),
            pl.BlockSpec((_D, _D), lambda i: (0, 0)),
            pl.BlockSpec((_D, 1), lambda i: (0, 0)),
            pl.BlockSpec((1, 1), lambda i: (0, 0)),
        ],
        out_specs=pl.BlockSpec((_BLK, 1), lambda i: (i, 0)),
        out_shape=jax.ShapeDtypeStruct((_N, 1), jnp.float32),
    )(x, a0, a1, cnt, W_l, b_l.reshape(1, _D), W_r, W_lin,
      b_lin.reshape(1, 1))


def kernel(x, edge_index, W_l, b_l, W_r, W_lin, b_lin):
    x2 = x
    pad = _EPAD - _E
    src = jnp.concatenate([edge_index[0], jnp.zeros((pad,), jnp.int32)])
    dst = jnp.concatenate([edge_index[1],
                           jnp.full((pad,), _TRASH, jnp.int32)])
    edges = jnp.concatenate([src, dst])
    zagg = jnp.zeros((_NPAD, _HALF), jnp.float32)
    zcnt = jnp.zeros((_NPAD,), jnp.float32)
    ones_h = jnp.ones((_CHUNK,), jnp.float32)
    agg, cnt = _sc_call(x2, edges, zagg, zcnt, ones_h)
    return _tc_tail(x, agg[:_NPAD], agg[_NPAD:], cnt[:_NPAD].reshape(_NPAD, 1),
                    W_l, b_l, W_r, W_lin, b_lin)


# speedup vs baseline: 1.3348x; 1.3348x over previous
import functools

import jax
import jax.numpy as jnp
from jax import lax
from jax.experimental import pallas as pl
from jax.experimental.pallas import tpu as pltpu
from jax.experimental.pallas import tpu_sc as plsc

_N = 10000
_E = 160000
_D = 256
_HALF = _D // 2
_CHUNK = 64
_NSUB = 16
_NCORE = 2
_CPT = 160
_ECHUNKS = _CPT * _NSUB
_EPAD = _ECHUNKS * _CHUNK
_NPAD = 10240
_TRASH = _NPAD - 8
_STRIPE = _NPAD // _NSUB


def _sc_body(x2, edges, zagg, zcnt, ones_h,
             agg_out, cnt_out,
             src_v, dst_v, rows_v, ones_v, agg_sh, cnt_sh,
             sem_g0, sem_g1, sem_g2, sem_g3,
             sem_d0, sem_d1, sem_d2, sem_d3):
    sem_g = (sem_g0, sem_g1, sem_g2, sem_g3)
    sem_d = (sem_d0, sem_d1, sem_d2, sem_d3)
    c = lax.axis_index("c")
    s = lax.axis_index("s")
    row0 = s * _STRIPE
    stripe = pl.ds(row0, _STRIPE)
    e0 = s * _CPT * _CHUNK

    pltpu.sync_copy(zagg.at[stripe], agg_sh.at[stripe])
    pltpu.sync_copy(zcnt.at[stripe], cnt_sh.at[stripe])
    pltpu.sync_copy(ones_h, ones_v)
    pltpu.sync_copy(edges.at[pl.ds(e0, _CPT * _CHUNK)], src_v)

    def xform(r, carry):
        sl = pl.ds(r * 16, 16)
        src_v[sl] = src_v[sl] * 2 + c
        return carry

    lax.fori_loop(0, _CPT * _CHUNK // 16, xform, 0)
    plsc.subcore_barrier()

    def gather(k, slot):
        return pltpu.make_async_copy(
            x2.at[src_v.at[pl.ds(k * _CHUNK, _CHUNK)]], rows_v.at[slot],
            sem_g[slot])

    def dst_load(k, slot):
        return pltpu.make_async_copy(
            edges.at[pl.ds(_EPAD + e0 + k * _CHUNK, _CHUNK)],
            dst_v.at[slot], sem_d[slot])

    def start(k, slot):
        dst_load(k, slot).start()
        gather(k, slot).start()

    def service(k, slot):
        gather(k, slot).wait()
        dst_load(k, slot).wait()
        pltpu.sync_copy(rows_v.at[slot], agg_sh.at[dst_v.at[slot]], add=True)
        pltpu.sync_copy(ones_v, cnt_sh.at[dst_v.at[slot]], add=True)

    start(0, 0)
    start(1, 1)
    start(2, 2)
    start(3, 3)
    service(0, 0)

    def group(g, carry):
        for u in range(4):
            k = 4 * g + u
            start(k, u)
            service(k - 3, (u + 1) % 4)
        return carry

    lax.fori_loop(1, _CPT // 4, group, 0)
    service(_CPT - 3, (_CPT - 3) % 4)
    service(_CPT - 2, (_CPT - 2) % 4)
    service(_CPT - 1, (_CPT - 1) % 4)

    plsc.subcore_barrier()

    pltpu.sync_copy(agg_sh.at[stripe],
                    agg_out.at[pl.ds(c * _NPAD + row0, _STRIPE)])
    pltpu.sync_copy(cnt_sh.at[stripe],
                    cnt_out.at[pl.ds(c * _NPAD + row0, _STRIPE)])


_sc_call = functools.partial(
    pl.kernel,
    out_type=(
        jax.ShapeDtypeStruct((_NCORE * _NPAD, _HALF), jnp.float32),
        jax.ShapeDtypeStruct((_NCORE * _NPAD,), jnp.float32),
    ),
    mesh=plsc.VectorSubcoreMesh(core_axis_name="c", subcore_axis_name="s"),
    scratch_types=[
        pltpu.VMEM((_CPT * _CHUNK,), jnp.int32),
        pltpu.VMEM((4, _CHUNK), jnp.int32),
        pltpu.VMEM((4, _CHUNK, _HALF), jnp.float32),
        pltpu.VMEM((_CHUNK,), jnp.float32),
        pltpu.VMEM_SHARED((_NPAD, _HALF), jnp.float32),
        pltpu.VMEM_SHARED((_NPAD,), jnp.float32),
        pltpu.SemaphoreType.DMA,
        pltpu.SemaphoreType.DMA,
        pltpu.SemaphoreType.DMA,
        pltpu.SemaphoreType.DMA,
        pltpu.SemaphoreType.DMA,
        pltpu.SemaphoreType.DMA,
        pltpu.SemaphoreType.DMA,
        pltpu.SemaphoreType.DMA,
    ],
)(_sc_body)


_BLK = 512


def _tc_body(x_ref, a0_ref, a1_ref, cnt_ref, wl_ref, bl_ref, wr_ref,
             wlin_ref, blin_ref, o_ref):
    inv = 1.0 / jnp.maximum(cnt_ref[...], 1.0)
    m0 = a0_ref[...] * inv
    m1 = a1_ref[...] * inv
    h = (jnp.dot(m0, wl_ref[0:_HALF, :], preferred_element_type=jnp.float32)
         + jnp.dot(m1, wl_ref[_HALF:, :], preferred_element_type=jnp.float32)
         + jnp.dot(x_ref[...], wr_ref[...], preferred_element_type=jnp.float32)
         + bl_ref[...])
    h = jnp.maximum(h, 0.0)
    o_ref[...] = jnp.dot(h, wlin_ref[...],
                         preferred_element_type=jnp.float32) + blin_ref[...]


def _tc_tail(x, agg, cnt, W_l, b_l, W_r, W_lin, b_lin):
    grid = (_NPAD // _BLK,)
    nb = _NPAD // _BLK
    return pl.pallas_call(
        _tc_body,
        grid=grid,
        in_specs=[
            pl.BlockSpec((_BLK, _D), lambda i: (i, 0)),
            pl.BlockSpec((_BLK, _HALF), lambda i: (i, 0)),
            pl.BlockSpec((_BLK, _HALF), lambda i: (i + _NPAD // _BLK, 0)),
            pl.BlockSpec((_BLK, 1), lambda i: (i, 0)),
            pl.BlockSpec((_D, _D), lambda i: (0, 0)),
            pl.BlockSpec((1, _D), lambda i: (0, 0)),
            pl.BlockSpec((_D, _D), lambda i: (0, 0)),
            pl.BlockSpec((_D, 1), lambda i: (0, 0)),
            pl.BlockSpec((1, 1), lambda i: (0, 0)),
        ],
        out_specs=pl.BlockSpec((_BLK, 1), lambda i: (i, 0)),
        out_shape=jax.ShapeDtypeStruct((_NPAD, 1), jnp.float32),
    )(x, agg, agg, cnt, W_l, b_l.reshape(1, _D), W_r, W_lin,
      b_lin.reshape(1, 1))


def kernel(x, edge_index, W_l, b_l, W_r, W_lin, b_lin):
    x2 = x.reshape(_NCORE * _N, _HALF)
    pad = _EPAD - _E
    src = jnp.concatenate([edge_index[0], jnp.zeros((pad,), jnp.int32)])
    dst = jnp.concatenate([edge_index[1],
                           jnp.full((pad,), _TRASH, jnp.int32)])
    edges = jnp.concatenate([src, dst])
    zagg = jnp.zeros((_NPAD, _HALF), jnp.float32)
    zcnt = jnp.zeros((_NPAD,), jnp.float32)
    ones_h = jnp.ones((_CHUNK,), jnp.float32)
    agg, cnt = _sc_call(x2, edges, zagg, zcnt, ones_h)
    out = _tc_tail(x, agg, cnt.reshape(_NCORE * _NPAD, 1),
                   W_l, b_l, W_r, W_lin, b_lin)
    return out[:_N]


# speedup vs baseline: 1.3489x; 1.0105x over previous
import functools

import jax
import jax.numpy as jnp
from jax import lax
from jax.experimental import pallas as pl
from jax.experimental.pallas import tpu as pltpu
from jax.experimental.pallas import tpu_sc as plsc

_N = 10000
_E = 160000
_D = 256
_HALF = _D // 2
_CHUNK = 64
_NSUB = 16
_NCORE = 2
_CPT = 160
_ECHUNKS = _CPT * _NSUB
_EPAD = _ECHUNKS * _CHUNK
_NPAD = 10240
_TRASH = _NPAD - 8
_STRIPE = _NPAD // _NSUB


def _sc_body(x2, edges, zagg, zcnt, ones_h,
             agg_out, cnt_out,
             src_v, dst_v, rows_v, ones_v, agg_sh, cnt_sh,
             sem_g0, sem_g1, sem_g2, sem_g3,
             sem_d0, sem_d1, sem_d2, sem_d3):
    sem_g = (sem_g0, sem_g1, sem_g2, sem_g3)
    sem_d = (sem_d0, sem_d1, sem_d2, sem_d3)
    c = lax.axis_index("c")
    s = lax.axis_index("s")
    row0 = s * _STRIPE
    stripe = pl.ds(row0, _STRIPE)
    e0 = s * _CPT * _CHUNK

    pltpu.sync_copy(zagg.at[stripe], agg_sh.at[stripe])
    pltpu.sync_copy(zcnt.at[stripe], cnt_sh.at[stripe])
    pltpu.sync_copy(ones_h, ones_v)
    pltpu.sync_copy(edges.at[pl.ds(e0, _CPT * _CHUNK)], src_v)

    def xform(r, carry):
        sl = pl.ds(r * 16, 16)
        src_v[sl] = src_v[sl] * 2 + c
        return carry

    lax.fori_loop(0, _CPT * _CHUNK // 16, xform, 0)
    plsc.subcore_barrier()

    def gather(k, slot):
        return pltpu.make_async_copy(
            x2.at[src_v.at[pl.ds(k * _CHUNK, _CHUNK)]], rows_v.at[slot],
            sem_g[slot])

    def dst_load(k, slot):
        return pltpu.make_async_copy(
            edges.at[pl.ds(_EPAD + e0 + k * _CHUNK, _CHUNK)],
            dst_v.at[slot], sem_d[slot])

    def start(k, slot):
        dst_load(k, slot).start()
        gather(k, slot).start()

    def service(k, slot):
        gather(k, slot).wait()
        dst_load(k, slot).wait()
        pltpu.sync_copy(rows_v.at[slot], agg_sh.at[dst_v.at[slot]], add=True)
        pltpu.sync_copy(ones_v, cnt_sh.at[dst_v.at[slot]], add=True)

    start(0, 0)
    start(1, 1)
    start(2, 2)
    start(3, 3)
    service(0, 0)

    def group(g, carry):
        for u in range(4):
            k = 4 * g + u
            start(k, u)
            service(k - 3, (u + 1) % 4)
        return carry

    lax.fori_loop(1, _CPT // 4, group, 0)
    service(_CPT - 3, (_CPT - 3) % 4)
    service(_CPT - 2, (_CPT - 2) % 4)
    service(_CPT - 1, (_CPT - 1) % 4)

    plsc.subcore_barrier()

    pltpu.sync_copy(agg_sh.at[stripe],
                    agg_out.at[pl.ds(c * _NPAD + row0, _STRIPE)])
    pltpu.sync_copy(cnt_sh.at[stripe],
                    cnt_out.at[pl.ds(c * _NPAD + row0, _STRIPE)])


_sc_call = functools.partial(
    pl.kernel,
    out_type=(
        jax.ShapeDtypeStruct((_NCORE * _NPAD, _HALF), jnp.float32),
        jax.ShapeDtypeStruct((_NCORE * _NPAD,), jnp.float32),
    ),
    mesh=plsc.VectorSubcoreMesh(core_axis_name="c", subcore_axis_name="s"),
    scratch_types=[
        pltpu.VMEM((_CPT * _CHUNK,), jnp.int32),
        pltpu.VMEM((4, _CHUNK), jnp.int32),
        pltpu.VMEM((4, _CHUNK, _HALF), jnp.float32),
        pltpu.VMEM((_CHUNK,), jnp.float32),
        pltpu.VMEM_SHARED((_NPAD, _HALF), jnp.float32),
        pltpu.VMEM_SHARED((_NPAD,), jnp.float32),
        pltpu.SemaphoreType.DMA,
        pltpu.SemaphoreType.DMA,
        pltpu.SemaphoreType.DMA,
        pltpu.SemaphoreType.DMA,
        pltpu.SemaphoreType.DMA,
        pltpu.SemaphoreType.DMA,
        pltpu.SemaphoreType.DMA,
        pltpu.SemaphoreType.DMA,
    ],
)(_sc_body)


_BLK = 1000


def _tc_body(x_ref, a0_ref, a1_ref, cnt_ref, wl_ref, bl_ref, wr_ref,
             wlin_ref, blin_ref, o_ref):
    inv = 1.0 / jnp.maximum(cnt_ref[...], 1.0)
    m0 = a0_ref[...] * inv
    m1 = a1_ref[...] * inv
    h = (jnp.dot(m0, wl_ref[0:_HALF, :], preferred_element_type=jnp.float32)
         + jnp.dot(m1, wl_ref[_HALF:, :], preferred_element_type=jnp.float32)
         + jnp.dot(x_ref[...], wr_ref[...], preferred_element_type=jnp.float32)
         + bl_ref[...])
    h = jnp.maximum(h, 0.0)
    o_ref[...] = jnp.dot(h, wlin_ref[...],
                         preferred_element_type=jnp.float32) + blin_ref[...]


def _tc_tail(x, a0, a1, cnt, W_l, b_l, W_r, W_lin, b_lin):
    grid = (_N // _BLK,)
    return pl.pallas_call(
        _tc_body,
        grid=grid,
        in_specs=[
            pl.BlockSpec((_BLK, _D), lambda i: (i, 0)),
            pl.BlockSpec((_BLK, _HALF), lambda i: (i, 0)),
            pl.BlockSpec((_BLK, _HALF), lambda i: (i, 0)),
            pl.BlockSpec((_BLK, 1), lambda i: (i, 0)),
            pl.BlockSpec((_D, _D), lambda i: (0, 0)),
            pl.BlockSpec((1, _D), lambda i: (0, 0)),
            pl.BlockSpec((_D, _D), lambda i: (0, 0)),
            pl.BlockSpec((_D, 1), lambda i: (0, 0)),
            pl.BlockSpec((1, 1), lambda i: (0, 0)),
        ],
        out_specs=pl.BlockSpec((_BLK, 1), lambda i: (i, 0)),
        out_shape=jax.ShapeDtypeStruct((_N, 1), jnp.float32),
    )(x, a0, a1, cnt, W_l, b_l.reshape(1, _D), W_r, W_lin,
      b_lin.reshape(1, 1))


def kernel(x, edge_index, W_l, b_l, W_r, W_lin, b_lin):
    x2 = x.reshape(_NCORE * _N, _HALF)
    pad = _EPAD - _E
    src = jnp.concatenate([edge_index[0], jnp.zeros((pad,), jnp.int32)])
    dst = jnp.concatenate([edge_index[1],
                           jnp.full((pad,), _TRASH, jnp.int32)])
    edges = jnp.concatenate([src, dst])
    zagg = jnp.zeros((_NPAD, _HALF), jnp.float32)
    zcnt = jnp.zeros((_NPAD,), jnp.float32)
    ones_h = jnp.ones((_CHUNK,), jnp.float32)
    agg, cnt = _sc_call(x2, edges, zagg, zcnt, ones_h)
    return _tc_tail(x, agg[:_NPAD], agg[_NPAD:], cnt[:_NPAD].reshape(_NPAD, 1),
                    W_l, b_l, W_r, W_lin, b_lin)
